# Initial kernel scaffold; baseline (speedup 1.0000x reference)
import jax
import jax.numpy as jnp
from jax.experimental import pallas as pl


def kernel(x, boundary, edge_index, relation_weight, W, b, gamma, beta):
    raise NotImplementedError("write your pallas kernel here")


# speedup vs baseline: 3.2902x; 3.2902x over previous
import functools

import jax
import jax.numpy as jnp
from jax import lax
from jax.experimental import pallas as pl
from jax.experimental.pallas import tpu as pltpu
from jax.experimental.pallas import tpu_sc as plsc

N = 10000
E = 320000
D = 128

NC = 2
NS = 16
NW = NC * NS
B = 128
K = 80
E_PAD = NW * K * B
NPAD = NW * 320


def _sc_agg_body(x_hbm, src_hbm, dst_hbm, z_hbm, out_hbm,
                 src_v, dst_v, rows_v, acc_sh, sem):
    c = lax.axis_index("c")
    s = lax.axis_index("s")
    wid = c * NS + s
    pltpu.sync_copy(z_hbm, acc_sh.at[pl.ds(s * 640, 640)])
    pltpu.sync_copy(src_hbm.at[pl.ds(wid * K, K)], src_v)
    pltpu.sync_copy(dst_hbm.at[pl.ds(wid * K, K)], dst_v)
    plsc.subcore_barrier()

    def step(j, carry):
        pltpu.async_copy(x_hbm.at[src_v.at[j]], rows_v, sem).wait()
        pltpu.sync_copy(rows_v, acc_sh.at[dst_v.at[j]], add=True)
        return carry

    lax.fori_loop(0, K, step, 0)
    plsc.subcore_barrier()
    pltpu.sync_copy(acc_sh.at[pl.ds(s * 640, 640)],
                    out_hbm.at[c, pl.ds(s * 640, 640)])


_sc_agg = pl.kernel(
    _sc_agg_body,
    mesh=plsc.VectorSubcoreMesh(core_axis_name="c", subcore_axis_name="s"),
    out_type=jax.ShapeDtypeStruct((NC, NPAD, D), jnp.float32),
    scratch_types=[
        pltpu.VMEM((K, B), jnp.int32),
        pltpu.VMEM((K, B), jnp.int32),
        pltpu.VMEM((B, D), jnp.float32),
        pltpu.VMEM_SHARED((NPAD, D), jnp.float32),
        pltpu.SemaphoreType.DMA,
    ],
)


def _tc_body(x_b, a0_b, a1_b, bnd_b, rw_b, w1_b, w2_b, bias_b, g_b, be_b, o_b):
    h2 = (a0_b[...] + a1_b[...]) * rw_b[...] + bnd_b[...]
    acc = jnp.dot(x_b[...], w1_b[...], preferred_element_type=jnp.float32)
    acc = acc + jnp.dot(h2, w2_b[...], preferred_element_type=jnp.float32)
    acc = acc + bias_b[...]
    mu = jnp.mean(acc, axis=1, keepdims=True)
    var = jnp.mean(jnp.square(acc - mu), axis=1, keepdims=True)
    y = (acc - mu) * lax.rsqrt(var + 1e-5)
    y = y * g_b[...] + be_b[...]
    o_b[...] = jnp.maximum(y, 0.0)


_R = 400


def _tc_stage(x, agg0, agg1, boundary, rw, w1t, w2t, bias, gamma, beta):
    row_spec = pl.BlockSpec((_R, D), lambda i: (i, 0))
    full_spec = pl.BlockSpec((D, D), lambda i: (0, 0))
    vec_spec = pl.BlockSpec((1, D), lambda i: (0, 0))
    return pl.pallas_call(
        _tc_body,
        grid=(N // _R,),
        in_specs=[row_spec, row_spec, row_spec, row_spec,
                  vec_spec, full_spec, full_spec, vec_spec, vec_spec, vec_spec],
        out_specs=row_spec,
        out_shape=jax.ShapeDtypeStruct((N, D), jnp.float32),
    )(x, agg0, agg1, boundary, rw, w1t, w2t, bias, gamma, beta)


def kernel(x, boundary, edge_index, relation_weight, W, b, gamma, beta):
    src = edge_index[0]
    dst = edge_index[1]
    pad = E_PAD - E
    src_p = jnp.concatenate([src, jnp.zeros((pad,), jnp.int32)]).reshape(NW * K, B)
    dst_p = jnp.concatenate([dst, jnp.full((pad,), N, jnp.int32)]).reshape(NW * K, B)
    zeros = jnp.zeros((640, D), jnp.float32)

    agg = _sc_agg(x, src_p, dst_p, zeros)

    w1t = W[:, :D].T
    w2t = W[:, D:].T
    return _tc_stage(x, agg[0, :N], agg[1, :N], boundary,
                     relation_weight.reshape(1, D), w1t, w2t,
                     b.reshape(1, D), gamma.reshape(1, D), beta.reshape(1, D))


# speedup vs baseline: 3.5141x; 1.0680x over previous
import functools

import jax
import jax.numpy as jnp
from jax import lax
from jax.experimental import pallas as pl
from jax.experimental.pallas import tpu as pltpu
from jax.experimental.pallas import tpu_sc as plsc

N = 10000
E = 320000
D = 128

NC = 2
NS = 16
NW = NC * NS
B = 128
K = 80
KH = K // 2
E_PAD = NW * K * B
NPAD = 10112
ROWS_T = NPAD // NS


def _sc_agg_body(x_hbm, src_hbm, dst_hbm, z_hbm, out_hbm,
                 src_v, dst_v, rows0, rows1, acc_sh, sem0, sem1):
    c = lax.axis_index("c")
    s = lax.axis_index("s")
    wid = c * NS + s
    pltpu.sync_copy(z_hbm, acc_sh.at[pl.ds(s * ROWS_T, ROWS_T)])
    plsc.subcore_barrier()

    for h in range(2):
        pltpu.sync_copy(src_hbm.at[pl.ds(wid * K + h * KH, KH)], src_v)
        pltpu.sync_copy(dst_hbm.at[pl.ds(wid * K + h * KH, KH)], dst_v)
        pltpu.async_copy(x_hbm.at[src_v.at[0]], rows0, sem0)

        def step(i, carry):
            j0 = 2 * i
            j1 = j0 + 1
            pltpu.make_async_copy(x_hbm.at[src_v.at[j0]], rows0, sem0).wait()
            pltpu.async_copy(x_hbm.at[src_v.at[j1]], rows1, sem1)
            pltpu.sync_copy(rows0, acc_sh.at[dst_v.at[j0]], add=True)
            pltpu.make_async_copy(x_hbm.at[src_v.at[j1]], rows1, sem1).wait()

            @pl.when(i < KH // 2 - 1)
            def _():
                pltpu.async_copy(x_hbm.at[src_v.at[j0 + 2]], rows0, sem0)

            pltpu.sync_copy(rows1, acc_sh.at[dst_v.at[j1]], add=True)
            return carry

        lax.fori_loop(0, KH // 2, step, 0)
    plsc.subcore_barrier()
    pltpu.sync_copy(acc_sh.at[pl.ds(s * ROWS_T, ROWS_T)],
                    out_hbm.at[c, pl.ds(s * ROWS_T, ROWS_T)])


_sc_agg = pl.kernel(
    _sc_agg_body,
    mesh=plsc.VectorSubcoreMesh(core_axis_name="c", subcore_axis_name="s"),
    out_type=jax.ShapeDtypeStruct((NC, NPAD, D), jnp.float32),
    scratch_types=[
        pltpu.VMEM((KH, B), jnp.int32),
        pltpu.VMEM((KH, B), jnp.int32),
        pltpu.VMEM((B, D), jnp.float32),
        pltpu.VMEM((B, D), jnp.float32),
        pltpu.VMEM_SHARED((NPAD, D), jnp.float32),
        pltpu.SemaphoreType.DMA,
        pltpu.SemaphoreType.DMA,
    ],
)


def _tc_body(x_b, a0_b, a1_b, bnd_b, rw_b, w1_b, w2_b, bias_b, g_b, be_b, o_b):
    h2 = (a0_b[...] + a1_b[...]) * rw_b[...] + bnd_b[...]
    acc = jnp.dot(x_b[...], w1_b[...], preferred_element_type=jnp.float32)
    acc = acc + jnp.dot(h2, w2_b[...], preferred_element_type=jnp.float32)
    acc = acc + bias_b[...]
    mu = jnp.mean(acc, axis=1, keepdims=True)
    var = jnp.mean(jnp.square(acc - mu), axis=1, keepdims=True)
    y = (acc - mu) * lax.rsqrt(var + 1e-5)
    y = y * g_b[...] + be_b[...]
    o_b[...] = jnp.maximum(y, 0.0)


_R = 400


def _tc_stage(x, agg0, agg1, boundary, rw, w1t, w2t, bias, gamma, beta):
    row_spec = pl.BlockSpec((_R, D), lambda i: (i, 0))
    full_spec = pl.BlockSpec((D, D), lambda i: (0, 0))
    vec_spec = pl.BlockSpec((1, D), lambda i: (0, 0))
    return pl.pallas_call(
        _tc_body,
        grid=(N // _R,),
        in_specs=[row_spec, row_spec, row_spec, row_spec,
                  vec_spec, full_spec, full_spec, vec_spec, vec_spec, vec_spec],
        out_specs=row_spec,
        out_shape=jax.ShapeDtypeStruct((N, D), jnp.float32),
    )(x, agg0, agg1, boundary, rw, w1t, w2t, bias, gamma, beta)


def kernel(x, boundary, edge_index, relation_weight, W, b, gamma, beta):
    src = edge_index[0]
    dst = edge_index[1]
    pad = E_PAD - E
    src_p = jnp.concatenate([src, jnp.zeros((pad,), jnp.int32)]).reshape(NW * K, B)
    dst_p = jnp.concatenate([dst, jnp.full((pad,), N, jnp.int32)]).reshape(NW * K, B)
    zeros = jnp.zeros((ROWS_T, D), jnp.float32)

    agg = _sc_agg(x, src_p, dst_p, zeros)

    w1t = W[:, :D].T
    w2t = W[:, D:].T
    return _tc_stage(x, agg[0, :N], agg[1, :N], boundary,
                     relation_weight.reshape(1, D), w1t, w2t,
                     b.reshape(1, D), gamma.reshape(1, D), beta.reshape(1, D))
